# baseline (speedup 1.0000x reference)
import functools

import jax
import jax.numpy as jnp
from jax import lax
from jax.experimental import pallas as pl
from jax.experimental.pallas import tpu as pltpu
from jax.experimental.pallas import tpu_sc as plsc

N_EXPERTS = 64
TOP_K = 8
N_TOKENS = 32768
BLOCK_R = 4096
GRID = N_TOKENS // BLOCK_R

_NC = 2
_NS = 16
_NW = _NC * _NS
_CHUNK = (TOP_K * N_TOKENS) // _NW
_ROWS_PER_SLOT = _NW // TOP_K


def _sc_hist_body(ids_hbm, out_hbm, idx_v, ones_v, part_v, shared):
    cid = lax.axis_index("c")
    sid = lax.axis_index("s")
    wid = sid * _NC + cid
    pltpu.sync_copy(ids_hbm.at[pl.ds(wid * _CHUNK, _CHUNK)], idx_v)

    zeros = jnp.zeros((16,), jnp.float32)
    ones = jnp.ones((16,), jnp.float32)

    @pl.loop(0, _CHUNK // 16)
    def _fill(i):
        ones_v[pl.ds(i * 16, 16)] = ones

    @pl.when(sid == 0)
    def _():
        @pl.loop(0, N_EXPERTS // 16)
        def _z(i):
            part_v[pl.ds(i * 16, 16)] = zeros
        pltpu.sync_copy(part_v, shared)

    plsc.subcore_barrier()
    pltpu.sync_copy(ones_v, shared.at[idx_v], add=True)
    plsc.subcore_barrier()

    @pl.when(sid == 0)
    def _():
        pltpu.sync_copy(shared, part_v)
        pltpu.sync_copy(part_v, out_hbm.at[pl.ds(cid * N_EXPERTS, N_EXPERTS)])


_sc_hist = pl.kernel(
    _sc_hist_body,
    mesh=plsc.VectorSubcoreMesh(core_axis_name="c", subcore_axis_name="s"),
    out_type=jax.ShapeDtypeStruct((_NC * N_EXPERTS,), jnp.float32),
    scratch_types=[
        pltpu.VMEM((_CHUNK,), jnp.int32),
        pltpu.VMEM((_CHUNK,), jnp.float32),
        pltpu.VMEM((N_EXPERTS,), jnp.float32),
        pltpu.VMEM_SHARED((N_EXPERTS,), jnp.float32),
    ],
)


def _router_body(x_ref, w_ref, ids_ref):
    x = x_ref[...].T
    iota0 = lax.broadcasted_iota(jnp.int32, (N_EXPERTS, BLOCK_R), 0)
    neg_inf = jnp.float32(-jnp.inf)

    vals = []
    ids = []
    for _ in range(TOP_K):
        m = jnp.max(x, axis=0, keepdims=True)
        cand = jnp.where(x == m, iota0, N_EXPERTS)
        idx = jnp.min(cand, axis=0, keepdims=True)
        vals.append(m)
        ids.append(idx)
        x = jnp.where(iota0 == idx, neg_inf, x)

    v8 = jnp.concatenate(vals, axis=0)
    i8 = jnp.concatenate(ids, axis=0)
    e = jnp.exp(v8 - v8[0:1, :])
    w_ref[...] = e / jnp.sum(e, axis=0, keepdims=True)
    ids_ref[...] = i8


@functools.partial(jax.jit)
def kernel(logits):
    w8, ids8 = pl.pallas_call(
        _router_body,
        grid=(GRID,),
        in_specs=[pl.BlockSpec((BLOCK_R, N_EXPERTS), lambda i: (i, 0))],
        out_specs=[
            pl.BlockSpec((TOP_K, BLOCK_R), lambda i: (0, i)),
            pl.BlockSpec((TOP_K, BLOCK_R), lambda i: (0, i)),
        ],
        out_shape=[
            jax.ShapeDtypeStruct((TOP_K, N_TOKENS), jnp.float32),
            jax.ShapeDtypeStruct((TOP_K, N_TOKENS), jnp.int32),
        ],
    )(logits)
    hist = _sc_hist(ids8.reshape(-1)).reshape(_NC, N_EXPERTS).sum(axis=0)
    return (logits, w8.T, ids8.T, hist)


# speedup vs baseline: 1.0360x; 1.0360x over previous
import functools

import jax
import jax.numpy as jnp
from jax import lax
from jax.experimental import pallas as pl
from jax.experimental.pallas import tpu as pltpu
from jax.experimental.pallas import tpu_sc as plsc

N_EXPERTS = 64
TOP_K = 8
N_TOKENS = 32768
BLOCK_R = 4096
GRID = N_TOKENS // BLOCK_R

_NC = 2
_NS = 16
_NW = _NC * _NS
_CHUNK = (TOP_K * N_TOKENS) // _NW
_ROWS_PER_SLOT = _NW // TOP_K


def _sc_hist_body(ids_hbm, out_hbm, idx_v, ones_v, part_v, shared):
    cid = lax.axis_index("c")
    sid = lax.axis_index("s")
    wid = sid * _NC + cid
    slot = wid // _ROWS_PER_SLOT
    base = (wid % _ROWS_PER_SLOT) * _CHUNK
    pltpu.sync_copy(ids_hbm.at[slot, pl.ds(base, _CHUNK)], idx_v)

    zeros = jnp.zeros((16,), jnp.float32)
    ones = jnp.ones((16,), jnp.float32)

    @pl.loop(0, _CHUNK // 16)
    def _fill(i):
        ones_v[pl.ds(i * 16, 16)] = ones

    @pl.when(sid == 0)
    def _():
        @pl.loop(0, N_EXPERTS // 16)
        def _z(i):
            part_v[pl.ds(i * 16, 16)] = zeros
        pltpu.sync_copy(part_v, shared)

    plsc.subcore_barrier()
    pltpu.sync_copy(ones_v, shared.at[idx_v], add=True)
    plsc.subcore_barrier()

    @pl.when(sid == 0)
    def _():
        pltpu.sync_copy(shared, part_v)
        pltpu.sync_copy(part_v, out_hbm.at[pl.ds(cid * N_EXPERTS, N_EXPERTS)])


_sc_hist = pl.kernel(
    _sc_hist_body,
    mesh=plsc.VectorSubcoreMesh(core_axis_name="c", subcore_axis_name="s"),
    out_type=jax.ShapeDtypeStruct((_NC * N_EXPERTS,), jnp.float32),
    scratch_types=[
        pltpu.VMEM((_CHUNK,), jnp.int32),
        pltpu.VMEM((_CHUNK,), jnp.float32),
        pltpu.VMEM((N_EXPERTS,), jnp.float32),
        pltpu.VMEM_SHARED((N_EXPERTS,), jnp.float32),
    ],
)


def _router_body(x_ref, w_ref, ids_ref):
    x = x_ref[...].T
    iota0 = lax.broadcasted_iota(jnp.int32, (N_EXPERTS, BLOCK_R), 0)
    neg_inf = jnp.float32(-jnp.inf)

    vals = []
    ids = []
    for _ in range(TOP_K):
        m = jnp.max(x, axis=0, keepdims=True)
        cand = jnp.where(x == m, iota0, N_EXPERTS)
        idx = jnp.min(cand, axis=0, keepdims=True)
        vals.append(m)
        ids.append(idx)
        x = jnp.where(iota0 == idx, neg_inf, x)

    v8 = jnp.concatenate(vals, axis=0)
    i8 = jnp.concatenate(ids, axis=0)
    e = jnp.exp(v8 - v8[0:1, :])
    w_ref[...] = e / jnp.sum(e, axis=0, keepdims=True)
    ids_ref[...] = i8


@functools.partial(jax.jit)
def kernel(logits):
    w8, ids8 = pl.pallas_call(
        _router_body,
        grid=(GRID,),
        in_specs=[pl.BlockSpec((BLOCK_R, N_EXPERTS), lambda i: (i, 0))],
        out_specs=[
            pl.BlockSpec((TOP_K, BLOCK_R), lambda i: (0, i)),
            pl.BlockSpec((TOP_K, BLOCK_R), lambda i: (0, i)),
        ],
        out_shape=[
            jax.ShapeDtypeStruct((TOP_K, N_TOKENS), jnp.float32),
            jax.ShapeDtypeStruct((TOP_K, N_TOKENS), jnp.int32),
        ],
    )(logits)
    hist = _sc_hist(ids8).reshape(_NC, N_EXPERTS).sum(axis=0)
    return (logits, w8.T, ids8.T, hist)
